# baseline (speedup 1.0000x reference)
import dataclasses
import functools

import jax
import jax.numpy as jnp
from jax import lax
from jax.experimental import pallas as pl
from jax.experimental.pallas import tpu as pltpu
from jax.experimental.pallas import tpu_sc as plsc

NC = 2
NS = 16
NW = NC * NS
LANES = 16

N = 10000
E = 320000
D = 128
R = 8

NPAD = 10240
RPS = NPAD // NS
NRP = 81920
CPS = NRP // NS
DH = D // 2
DQ = D // 4
NQ = 2
EPW = E // NW
EPT = E // NS
K = 80
NCHUNK = EPW // K
ACHUNK = EPT // K
ZB = 640
SS = 10
SC5 = 5

_mesh = plsc.VectorSubcoreMesh(core_axis_name="c", subcore_axis_name="s")

_sc_params = pltpu.CompilerParams()
for _f, _v in (("needs_layout_passes", False), ("use_tc_tiling_on_sc", False)):
  if _f in pltpu.CompilerParams.__dataclass_fields__:
    _sc_params = dataclasses.replace(_sc_params, **{_f: _v})


def _tc_transforms(x, w):
  bn = 2000
  nb = N // bn

  def body(x_ref, w_ref, o_ref):
    res = jnp.dot(x_ref[...], w_ref[0], preferred_element_type=jnp.float32)
    for qq in range(NC * NQ):
      o_ref[qq] = res[:, qq * DQ:(qq + 1) * DQ]

  return pl.pallas_call(
      body,
      grid=(nb, R),
      in_specs=[
          pl.BlockSpec((bn, D), lambda i, r: (i, 0)),
          pl.BlockSpec((1, D, D), lambda i, r: (r, 0, 0)),
      ],
      out_specs=pl.BlockSpec((NC * NQ, bn, DQ), lambda i, r: (0, r * nb + i, 0)),
      out_shape=jax.ShapeDtypeStruct((NC * NQ, R * N, DQ), jnp.float32),
  )(x, w)


def _tc_combine(x, root, bias, p, relu):
  bn = 2000
  nb = N // bn

  def body(x_ref, r_ref, b_ref, p_ref, o_ref):
    acc = jnp.dot(x_ref[...], r_ref[...], preferred_element_type=jnp.float32)
    acc = acc + b_ref[...] + jnp.concatenate(
        [p_ref[0, 0], p_ref[0, 1], p_ref[1, 0], p_ref[1, 1]], axis=-1)
    if relu:
      acc = jnp.maximum(acc, 0.0)
    o_ref[...] = acc

  return pl.pallas_call(
      body,
      grid=(nb,),
      in_specs=[
          pl.BlockSpec((bn, D), lambda i: (i, 0)),
          pl.BlockSpec((D, D), lambda i: (0, 0)),
          pl.BlockSpec((1, D), lambda i: (0, 0)),
          pl.BlockSpec((NC, NQ, bn, DQ), lambda i: (0, 0, i, 0)),
      ],
      out_specs=pl.BlockSpec((bn, D), lambda i: (i, 0)),
      out_shape=jax.ShapeDtypeStruct((N, D), jnp.float32),
  )(x, root, bias.reshape(1, D), p)


def _tc_inv(counts):
  def body(c_ref, o_ref):
    o_ref[...] = 1.0 / jnp.maximum(c_ref[0] + c_ref[1], 1.0)

  out = pl.pallas_call(
      body,
      out_shape=jax.ShapeDtypeStruct((NRP // D, D), jnp.float32),
  )(counts.reshape(NC, NRP // D, D))
  return out.reshape(NRP)


def _tc_combine_transforms(x, root, bias, p, w):
  bn = 2000
  nb = N // bn

  def body(x_ref, r_ref, b_ref, p_ref, w_ref, h_ref, o_ref):
    hblk = jnp.dot(x_ref[...], r_ref[...], preferred_element_type=jnp.float32)
    hblk = hblk + b_ref[...] + jnp.concatenate(
        [p_ref[0, 0], p_ref[0, 1], p_ref[1, 0], p_ref[1, 1]], axis=-1)
    hblk = jnp.maximum(hblk, 0.0)
    h_ref[...] = hblk
    res = jnp.dot(hblk, w_ref[0], preferred_element_type=jnp.float32)
    for qq in range(NC * NQ):
      o_ref[qq] = res[:, qq * DQ:(qq + 1) * DQ]

  return pl.pallas_call(
      body,
      grid=(nb, R),
      in_specs=[
          pl.BlockSpec((bn, D), lambda i, r: (i, 0)),
          pl.BlockSpec((D, D), lambda i, r: (0, 0)),
          pl.BlockSpec((1, D), lambda i, r: (0, 0)),
          pl.BlockSpec((NC, NQ, bn, DQ), lambda i, r: (0, 0, i, 0)),
          pl.BlockSpec((1, D, D), lambda i, r: (r, 0, 0)),
      ],
      out_specs=[
          pl.BlockSpec((bn, D), lambda i, r: (i, 0)),
          pl.BlockSpec((NC * NQ, bn, DQ), lambda i, r: (0, r * nb + i, 0)),
      ],
      out_shape=[
          jax.ShapeDtypeStruct((N, D), jnp.float32),
          jax.ShapeDtypeStruct((NC * NQ, R * N, DQ), jnp.float32),
      ],
  )(x, root, bias.reshape(1, D), p, w)


def _sc_counts(src2, dst2, typ2):
  @functools.partial(
      pl.kernel,
      out_type=(jax.ShapeDtypeStruct((NC, NRP), jnp.float32),
                jax.ShapeDtypeStruct((E,), jnp.int32)),
      mesh=_mesh,
      compiler_params=_sc_params,
      scratch_types=[
          pltpu.VMEM_SHARED((NRP,), jnp.float32),
          pltpu.VMEM((ZB,), jnp.float32),
          pltpu.VMEM((K,), jnp.float32),
          pltpu.VMEM((SC5, K), jnp.int32),
          pltpu.VMEM((SC5, K), jnp.int32),
          pltpu.VMEM((SC5, K), jnp.int32),
          pltpu.VMEM((1, K), jnp.int32),
          pltpu.VMEM((K,), jnp.int32),
          pltpu.SemaphoreType.DMA,
      ],
  )
  def k(src_hbm, dst_hbm, typ_hbm, counts_hbm, g_hbm,
        cacc, zb, ones, srcb, dstb, typb, cidxb, gb, sem):
    cid = lax.axis_index("c")
    sid = lax.axis_index("s")
    wid = cid * NS + sid

    @pl.loop(0, ZB, step=LANES)
    def _(i):
      zb[pl.ds(i, LANES)] = jnp.zeros((LANES,), jnp.float32)

    @pl.loop(0, K, step=LANES)
    def _(i):
      ones[pl.ds(i, LANES)] = jnp.ones((LANES,), jnp.float32)

    @pl.loop(0, CPS, step=ZB)
    def _(i):
      pltpu.sync_copy(zb, cacc.at[pl.ds(sid * CPS + i, ZB)])

    plsc.subcore_barrier()

    rbase0 = wid * NCHUNK

    @pl.loop(0, NCHUNK // SC5)
    def _(j):
      rbase = rbase0 + j * SC5
      pltpu.sync_copy(src_hbm.at[pl.ds(rbase, SC5)], srcb)
      pltpu.sync_copy(dst_hbm.at[pl.ds(rbase, SC5)], dstb)
      pltpu.sync_copy(typ_hbm.at[pl.ds(rbase, SC5)], typb)
      for s in range(SC5):
        for m in range(K // LANES):
          sl = pl.ds(m * LANES, LANES)
          t16 = typb[s, sl]
          gb[sl] = t16 * N + srcb[s, sl]
          cidxb[0, sl] = dstb[s, sl] * R + t16
        pltpu.sync_copy(gb, g_hbm.at[pl.ds((rbase + s) * K, K)])
        pltpu.sync_copy(ones, cacc.at[cidxb.at[0]], add=True)

    plsc.subcore_barrier()
    pltpu.sync_copy(cacc.at[pl.ds(sid * CPS, CPS)],
                    counts_hbm.at[cid, pl.ds(sid * CPS, CPS)])

  return k(src2, dst2, typ2)


def _sc_aggregate(yq, g2, dst2, typ2, inv):
  @functools.partial(
      pl.kernel,
      out_type=jax.ShapeDtypeStruct((NC, NQ, NS, RPS, DQ), jnp.float32),
      mesh=_mesh,
      compiler_params=_sc_params,
      scratch_types=[
          pltpu.VMEM_SHARED((NPAD, DQ), jnp.float32),
          pltpu.VMEM((NRP,), jnp.float32),
          pltpu.VMEM((4, K, DQ), jnp.float32),
          pltpu.VMEM((SS, K), jnp.int32),
          pltpu.VMEM((SS, K), jnp.int32),
          pltpu.VMEM((SS, K), jnp.int32),
          pltpu.SemaphoreType.DMA,
          pltpu.SemaphoreType.DMA,
          pltpu.SemaphoreType.DMA,
          pltpu.SemaphoreType.DMA,
          pltpu.SemaphoreType.DMA,
          pltpu.SemaphoreType.DMA,
          pltpu.SemaphoreType.DMA,
          pltpu.SemaphoreType.DMA,
      ],
  )
  def k(y_hbm, g_hbm, dst_hbm, typ_hbm, inv_hbm, out_hbm,
        acc, invb, rows2, sgb, sdstb, stypb,
        gsem0, gsem1, gsem2, gsem3, ssem0, ssem1, ssem2, ssem3):
    cid = lax.axis_index("c")
    sid = lax.axis_index("s")
    gsems = (gsem0, gsem1, gsem2, gsem3)
    ssems = (ssem0, ssem1, ssem2, ssem3)

    pltpu.sync_copy(inv_hbm, invb)

    def _zero_slice():
      for kk in range(K):
        for c in range(DQ // LANES):
          rows2[0, kk, pl.ds(c * LANES, LANES)] = jnp.zeros((LANES,),
                                                            jnp.float32)

      @pl.loop(0, RPS, step=K)
      def _(i):
        pltpu.sync_copy(rows2.at[0], acc.at[pl.ds(sid * RPS + i, K)])

    rbase0 = sid * (EPT // K)
    for q in range(NQ):
      quarter = cid * NQ + q
      yq_ref = y_hbm.at[quarter]
      _zero_slice()
      plsc.subcore_barrier()

      @pl.loop(0, ACHUNK // SS)
      def _(sc):
        rbase = rbase0 + sc * SS
        pltpu.sync_copy(g_hbm.at[pl.ds(rbase, SS)], sgb)
        pltpu.sync_copy(dst_hbm.at[pl.ds(rbase, SS)], sdstb)
        pltpu.sync_copy(typ_hbm.at[pl.ds(rbase, SS)], stypb)

        gd = [None] * 4
        sd = [None] * 4
        for p in range(3):
          gd[p] = pltpu.async_copy(yq_ref.at[sgb.at[p]], rows2.at[p],
                                   gsems[p])
        for s in range(SS):
          b = s % 4
          if s + 3 < SS:
            nb2 = (s + 3) % 4
            if sd[nb2] is not None:
              sd[nb2].wait()
              sd[nb2] = None
            gd[nb2] = pltpu.async_copy(yq_ref.at[sgb.at[s + 3]],
                                       rows2.at[nb2], gsems[nb2])
          gd[b].wait()
          rows = rows2.at[b]
          for m in range(K // LANES):
            sl = pl.ds(m * LANES, LANES)
            cidx = sdstb[s, sl] * R + stypb[s, sl]
            s16 = plsc.load_gather(invb, [cidx])
            for t in range(LANES):
              sp = s16.at[jnp.full((LANES,), t, jnp.int32)].get(
                  mode="promise_in_bounds")
              kk = m * LANES + t
              for c in range(DQ // LANES):
                sl2 = pl.ds(c * LANES, LANES)
                rows[kk, sl2] = rows[kk, sl2] * sp
          sd[b] = pltpu.async_copy(rows, acc.at[sdstb.at[s]], ssems[b],
                                   add=True)
        for bb in range(4):
          if sd[bb] is not None:
            sd[bb].wait()

      plsc.subcore_barrier()
      pltpu.sync_copy(acc.at[pl.ds(sid * RPS, RPS)], out_hbm.at[cid, q, sid])
      plsc.subcore_barrier()

  return k(yq, g2, dst2, typ2, inv)


def kernel(x, edge_index, edge_type, conv1_weight, conv1_root, conv1_bias,
           conv2_weight, conv2_root, conv2_bias):
  src = edge_index[0]
  dst = edge_index[1]
  typ = edge_type

  src2 = src.reshape(E // K, K)
  dst2 = dst.reshape(E // K, K)
  typ2 = typ.reshape(E // K, K)

  counts, g = _sc_counts(src2, dst2, typ2)
  inv = _tc_inv(counts)

  g2 = g.reshape(E // K, K)

  y1 = _tc_transforms(x, conv1_weight)
  p1 = _sc_aggregate(y1, g2, dst2, typ2, inv).reshape(NC, NQ, NPAD, DQ)
  h, y2 = _tc_combine_transforms(x, conv1_root, conv1_bias, p1, conv2_weight)
  p2 = _sc_aggregate(y2, g2, dst2, typ2, inv).reshape(NC, NQ, NPAD, DQ)
  out = _tc_combine(h, conv2_root, conv2_bias, p2, relu=False)
  return out


# speedup vs baseline: 1.0118x; 1.0118x over previous
import dataclasses
import functools

import jax
import jax.numpy as jnp
from jax import lax
from jax.experimental import pallas as pl
from jax.experimental.pallas import tpu as pltpu
from jax.experimental.pallas import tpu_sc as plsc

NC = 2
NS = 16
NW = NC * NS
LANES = 16

N = 10000
E = 320000
D = 128
R = 8

NPAD = 10240
RPS = NPAD // NS
NRP = 81920
CPS = NRP // NS
DH = D // 2
DQ = D // 4
NQ = 2
EPW = E // NW
EPT = E // NS
K = 80
NCHUNK = EPW // K
ACHUNK = EPT // K
ZB = 640
SS = 10
SC5 = 5

_mesh = plsc.VectorSubcoreMesh(core_axis_name="c", subcore_axis_name="s")

_sc_params = pltpu.CompilerParams()
for _f, _v in (("needs_layout_passes", False), ("use_tc_tiling_on_sc", False)):
  if _f in pltpu.CompilerParams.__dataclass_fields__:
    _sc_params = dataclasses.replace(_sc_params, **{_f: _v})


def _tc_transforms(x, w):
  bn = 2000
  nb = N // bn

  def body(x_ref, w_ref, o_ref):
    res = jnp.dot(x_ref[...], w_ref[0], preferred_element_type=jnp.float32)
    for qq in range(NC * NQ):
      o_ref[qq] = res[:, qq * DQ:(qq + 1) * DQ]

  return pl.pallas_call(
      body,
      grid=(nb, R),
      in_specs=[
          pl.BlockSpec((bn, D), lambda i, r: (i, 0)),
          pl.BlockSpec((1, D, D), lambda i, r: (r, 0, 0)),
      ],
      out_specs=pl.BlockSpec((NC * NQ, bn, DQ), lambda i, r: (0, r * nb + i, 0)),
      out_shape=jax.ShapeDtypeStruct((NC * NQ, R * N, DQ), jnp.float32),
  )(x, w)


def _tc_combine(x, root, bias, p, relu):
  bn = 2000
  nb = N // bn

  def body(x_ref, r_ref, b_ref, p_ref, o_ref):
    acc = jnp.dot(x_ref[...], r_ref[...], preferred_element_type=jnp.float32)
    acc = acc + b_ref[...] + jnp.concatenate(
        [p_ref[0, 0], p_ref[0, 1], p_ref[1, 0], p_ref[1, 1]], axis=-1)
    if relu:
      acc = jnp.maximum(acc, 0.0)
    o_ref[...] = acc

  return pl.pallas_call(
      body,
      grid=(nb,),
      in_specs=[
          pl.BlockSpec((bn, D), lambda i: (i, 0)),
          pl.BlockSpec((D, D), lambda i: (0, 0)),
          pl.BlockSpec((1, D), lambda i: (0, 0)),
          pl.BlockSpec((NC, NQ, bn, DQ), lambda i: (0, 0, i, 0)),
      ],
      out_specs=pl.BlockSpec((bn, D), lambda i: (i, 0)),
      out_shape=jax.ShapeDtypeStruct((N, D), jnp.float32),
  )(x, root, bias.reshape(1, D), p)


def _tc_inv(counts):
  def body(c_ref, o_ref):
    o_ref[...] = 1.0 / jnp.maximum(c_ref[0] + c_ref[1], 1.0)

  out = pl.pallas_call(
      body,
      out_shape=jax.ShapeDtypeStruct((NRP // D, D), jnp.float32),
  )(counts.reshape(NC, NRP // D, D))
  return out.reshape(NRP)


def _sc_counts(src2, dst2, typ2):
  @functools.partial(
      pl.kernel,
      out_type=(jax.ShapeDtypeStruct((NC, NRP), jnp.float32),
                jax.ShapeDtypeStruct((E,), jnp.int32)),
      mesh=_mesh,
      compiler_params=_sc_params,
      scratch_types=[
          pltpu.VMEM_SHARED((NRP,), jnp.float32),
          pltpu.VMEM((ZB,), jnp.float32),
          pltpu.VMEM((K,), jnp.float32),
          pltpu.VMEM((SC5, K), jnp.int32),
          pltpu.VMEM((SC5, K), jnp.int32),
          pltpu.VMEM((SC5, K), jnp.int32),
          pltpu.VMEM((1, K), jnp.int32),
          pltpu.VMEM((2, K), jnp.int32),
          pltpu.SemaphoreType.DMA,
          pltpu.SemaphoreType.DMA,
          pltpu.SemaphoreType.DMA,
      ],
  )
  def k(src_hbm, dst_hbm, typ_hbm, counts_hbm, g_hbm,
        cacc, zb, ones, srcb, dstb, typb, cidxb, gb2, sem, gwsem0, gwsem1):
    gwsems = (gwsem0, gwsem1)
    cid = lax.axis_index("c")
    sid = lax.axis_index("s")
    wid = cid * NS + sid

    @pl.loop(0, ZB, step=LANES)
    def _(i):
      zb[pl.ds(i, LANES)] = jnp.zeros((LANES,), jnp.float32)

    @pl.loop(0, K, step=LANES)
    def _(i):
      ones[pl.ds(i, LANES)] = jnp.ones((LANES,), jnp.float32)

    @pl.loop(0, CPS, step=ZB)
    def _(i):
      pltpu.sync_copy(zb, cacc.at[pl.ds(sid * CPS + i, ZB)])

    plsc.subcore_barrier()

    rbase0 = wid * NCHUNK

    @pl.loop(0, NCHUNK // SC5)
    def _(j):
      rbase = rbase0 + j * SC5
      pltpu.sync_copy(src_hbm.at[pl.ds(rbase, SC5)], srcb)
      pltpu.sync_copy(dst_hbm.at[pl.ds(rbase, SC5)], dstb)
      pltpu.sync_copy(typ_hbm.at[pl.ds(rbase, SC5)], typb)
      gw = [None, None]
      for s in range(SC5):
        b = s % 2
        if gw[b] is not None:
          gw[b].wait()
          gw[b] = None
        gb = gb2.at[b]
        for m in range(K // LANES):
          sl = pl.ds(m * LANES, LANES)
          t16 = typb[s, sl]
          gb[sl] = t16 * N + srcb[s, sl]
          cidxb[0, sl] = dstb[s, sl] * R + t16
        gw[b] = pltpu.async_copy(gb, g_hbm.at[pl.ds((rbase + s) * K, K)],
                                 gwsems[b])
        pltpu.sync_copy(ones, cacc.at[cidxb.at[0]], add=True)
      for b in range(2):
        if gw[b] is not None:
          gw[b].wait()

    plsc.subcore_barrier()
    pltpu.sync_copy(cacc.at[pl.ds(sid * CPS, CPS)],
                    counts_hbm.at[cid, pl.ds(sid * CPS, CPS)])

  return k(src2, dst2, typ2)


def _sc_aggregate(yq, g2, dst2, typ2, inv):
  @functools.partial(
      pl.kernel,
      out_type=jax.ShapeDtypeStruct((NC, NQ, NS, RPS, DQ), jnp.float32),
      mesh=_mesh,
      compiler_params=_sc_params,
      scratch_types=[
          pltpu.VMEM_SHARED((NPAD, DQ), jnp.float32),
          pltpu.VMEM((NRP,), jnp.float32),
          pltpu.VMEM((4, K, DQ), jnp.float32),
          pltpu.VMEM((SS, K), jnp.int32),
          pltpu.VMEM((SS, K), jnp.int32),
          pltpu.VMEM((SS, K), jnp.int32),
          pltpu.SemaphoreType.DMA,
          pltpu.SemaphoreType.DMA,
          pltpu.SemaphoreType.DMA,
          pltpu.SemaphoreType.DMA,
          pltpu.SemaphoreType.DMA,
          pltpu.SemaphoreType.DMA,
          pltpu.SemaphoreType.DMA,
          pltpu.SemaphoreType.DMA,
      ],
  )
  def k(y_hbm, g_hbm, dst_hbm, typ_hbm, inv_hbm, out_hbm,
        acc, invb, rows2, sgb, sdstb, stypb,
        gsem0, gsem1, gsem2, gsem3, ssem0, ssem1, ssem2, ssem3):
    cid = lax.axis_index("c")
    sid = lax.axis_index("s")
    gsems = (gsem0, gsem1, gsem2, gsem3)
    ssems = (ssem0, ssem1, ssem2, ssem3)

    pltpu.sync_copy(inv_hbm, invb)

    def _zero_slice():
      for kk in range(K):
        for c in range(DQ // LANES):
          rows2[0, kk, pl.ds(c * LANES, LANES)] = jnp.zeros((LANES,),
                                                            jnp.float32)

      @pl.loop(0, RPS, step=K)
      def _(i):
        pltpu.sync_copy(rows2.at[0], acc.at[pl.ds(sid * RPS + i, K)])

    rbase0 = sid * (EPT // K)
    for q in range(NQ):
      quarter = cid * NQ + q
      yq_ref = y_hbm.at[quarter]
      _zero_slice()
      plsc.subcore_barrier()

      @pl.loop(0, ACHUNK // SS)
      def _(sc):
        rbase = rbase0 + sc * SS
        pltpu.sync_copy(g_hbm.at[pl.ds(rbase, SS)], sgb)
        pltpu.sync_copy(dst_hbm.at[pl.ds(rbase, SS)], sdstb)
        pltpu.sync_copy(typ_hbm.at[pl.ds(rbase, SS)], stypb)

        gd = [None] * 4
        sd = [None] * 4
        for p in range(3):
          gd[p] = pltpu.async_copy(yq_ref.at[sgb.at[p]], rows2.at[p],
                                   gsems[p])
        for s in range(SS):
          b = s % 4
          if s + 3 < SS:
            nb2 = (s + 3) % 4
            if sd[nb2] is not None:
              sd[nb2].wait()
              sd[nb2] = None
            gd[nb2] = pltpu.async_copy(yq_ref.at[sgb.at[s + 3]],
                                       rows2.at[nb2], gsems[nb2])
          gd[b].wait()
          rows = rows2.at[b]
          for m in range(K // LANES):
            sl = pl.ds(m * LANES, LANES)
            cidx = sdstb[s, sl] * R + stypb[s, sl]
            s16 = plsc.load_gather(invb, [cidx])
            for t in range(LANES):
              sp = s16.at[jnp.full((LANES,), t, jnp.int32)].get(
                  mode="promise_in_bounds")
              kk = m * LANES + t
              for c in range(DQ // LANES):
                sl2 = pl.ds(c * LANES, LANES)
                rows[kk, sl2] = rows[kk, sl2] * sp
          sd[b] = pltpu.async_copy(rows, acc.at[sdstb.at[s]], ssems[b],
                                   add=True)
        for bb in range(4):
          if sd[bb] is not None:
            sd[bb].wait()

      plsc.subcore_barrier()
      pltpu.sync_copy(acc.at[pl.ds(sid * RPS, RPS)], out_hbm.at[cid, q, sid])
      plsc.subcore_barrier()

  return k(yq, g2, dst2, typ2, inv)


def kernel(x, edge_index, edge_type, conv1_weight, conv1_root, conv1_bias,
           conv2_weight, conv2_root, conv2_bias):
  src = edge_index[0]
  dst = edge_index[1]
  typ = edge_type

  src2 = src.reshape(E // K, K)
  dst2 = dst.reshape(E // K, K)
  typ2 = typ.reshape(E // K, K)

  counts, g = _sc_counts(src2, dst2, typ2)
  inv = _tc_inv(counts)

  g2 = g.reshape(E // K, K)

  y1 = _tc_transforms(x, conv1_weight)
  p1 = _sc_aggregate(y1, g2, dst2, typ2, inv).reshape(NC, NQ, NPAD, DQ)
  h = _tc_combine(x, conv1_root, conv1_bias, p1, relu=True)

  y2 = _tc_transforms(h, conv2_weight)
  p2 = _sc_aggregate(y2, g2, dst2, typ2, inv).reshape(NC, NQ, NPAD, DQ)
  out = _tc_combine(h, conv2_root, conv2_bias, p2, relu=False)
  return out
